# baseline (speedup 1.0000x reference)
import functools

import jax
import jax.numpy as jnp
from jax import lax
from jax.experimental import pallas as pl
from jax.experimental.pallas import tpu as pltpu

B = 64
P = 8732
C = 21
NT = 16
RB = 8
_THRESHOLD = 0.5
_NEGPOS_RATIO = 3
_V0 = 0.1
_V1 = 0.2


def _smooth_l1(x):
    ax = jnp.abs(x)
    return jnp.where(ax < 1.0, 0.5 * x * x, ax - 0.5)


def _stage_a1(pr_ref, tg_ref, tgt_ref, comb_ref):
    pr = pr_ref[...]
    t = tg_ref[0]
    t5 = tgt_ref[0]

    px1 = pr[0:1] - pr[2:3] * 0.5
    py1 = pr[1:2] - pr[3:4] * 0.5
    px2 = pr[0:1] + pr[2:3] * 0.5
    py2 = pr[1:2] + pr[3:4] * 0.5

    tx1 = t[:, 0:1]
    ty1 = t[:, 1:2]
    tx2 = t[:, 2:3]
    ty2 = t[:, 3:4]

    iw = jnp.maximum(jnp.minimum(tx2, px2) - jnp.maximum(tx1, px1), 0.0)
    ih = jnp.maximum(jnp.minimum(ty2, py2) - jnp.maximum(ty1, py1), 0.0)
    inter = iw * ih
    area_a = (tx2 - tx1) * (ty2 - ty1)
    area_b = (px2 - px1) * (py2 - py1)
    ov = inter / (area_a + area_b - inter)

    ti = lax.broadcasted_iota(jnp.int32, (NT, P), 0)
    li = lax.broadcasted_iota(jnp.int32, (NT, P), 1)

    bto = jnp.max(ov, axis=0, keepdims=True)
    bti = jnp.min(jnp.where(ov == bto, ti, NT), axis=0, keepdims=True)

    bpo = jnp.max(ov, axis=1, keepdims=True)
    bpi = jnp.min(jnp.where(ov == bpo, li, P), axis=1, keepdims=True)

    eq = li == bpi
    forced_t = jnp.max(jnp.where(eq, ti, -1), axis=0, keepdims=True)
    forced = forced_t >= 0
    bti2 = jnp.where(forced, forced_t, bti)
    bto2 = jnp.where(forced, 2.0, bto)

    oh = (ti == bti2).astype(jnp.float32)
    matched = jnp.dot(t5, oh, preferred_element_type=jnp.float32)

    comb_ref[0, 0:5, :] = matched
    comb_ref[0, 5:6, :] = bto2


def _stage_a2(loc_ref, conf_ref, comb_ref, pr_ref,
              ll_ref, lc_ref, np_ref):
    pr = pr_ref[...]
    cx = pr[0:1]
    cy = pr[1:2]
    w = pr[2:3]
    h = pr[3:4]

    mx1 = comb_ref[:, 0, :]
    my1 = comb_ref[:, 1, :]
    mx2 = comb_ref[:, 2, :]
    my2 = comb_ref[:, 3, :]
    mlab = comb_ref[:, 4, :]
    bto2 = comb_ref[:, 5, :]

    conf = jnp.where(bto2 < _THRESHOLD, 0, mlab.astype(jnp.int32) + 1)
    pos = conf > 0

    gx = ((mx1 + mx2) * 0.5 - cx) / (_V0 * w)
    gy = ((my1 + my2) * 0.5 - cy) / (_V0 * h)
    gw = jnp.log((mx2 - mx1) / w) / _V1
    gh = jnp.log((my2 - my1) / h) / _V1

    sl = (_smooth_l1(loc_ref[:, 0, :] - gx) + _smooth_l1(loc_ref[:, 1, :] - gy)
          + _smooth_l1(loc_ref[:, 2, :] - gw) + _smooth_l1(loc_ref[:, 3, :] - gh))
    loss_l = jnp.sum(jnp.where(pos, sl, 0.0))

    cf = conf_ref[...]
    m = jnp.max(cf, axis=1)
    s = jnp.sum(jnp.exp(cf - m[:, None, :]), axis=1)
    lse = jnp.log(s) + m
    ci = lax.broadcasted_iota(jnp.int32, (RB, C, P), 1)
    gat = jnp.sum(jnp.where(ci == conf[:, None, :], cf, 0.0), axis=1)
    ce = lse - gat

    cep = jnp.sum(jnp.where(pos, ce, 0.0))
    mined = jnp.maximum(jnp.where(pos, 0.0, ce), 0.0)
    npos = jnp.sum(pos.astype(jnp.float32), axis=1, keepdims=True)

    keys = lax.bitcast_convert_type(mined, jnp.int32)
    k = jnp.minimum(
        _NEGPOS_RATIO * npos.astype(jnp.int32),
        jnp.int32(P - 1))

    def body(i, v):
        bit = jnp.left_shift(jnp.int32(1), 30 - i)
        cand = v | bit
        cnt = jnp.sum((keys >= cand).astype(jnp.int32), axis=1, keepdims=True)
        return jnp.where(cnt >= k, cand, v)

    v = lax.fori_loop(0, 31, body, jnp.zeros((RB, 1), jnp.int32))

    gt = keys > v
    cnt_gt = jnp.sum(gt.astype(jnp.int32), axis=1, keepdims=True)
    vf = jnp.max(jnp.where(gt, 0.0, mined), axis=1, keepdims=True)
    topk = (jnp.sum(jnp.where(gt, mined, 0.0), axis=1, keepdims=True)
            + (k - cnt_gt).astype(jnp.float32) * vf)
    topk = jnp.where(k == 0, 0.0, topk)

    ll_ref[...] = jnp.broadcast_to(loss_l, (1, 1, 128))
    lc_ref[...] = jnp.broadcast_to(cep + jnp.sum(topk), (1, 1, 128))
    np_ref[...] = jnp.broadcast_to(jnp.sum(npos), (1, 1, 128))


@functools.partial(jax.jit, static_argnames=())
def kernel(loc_data, conf_data, priors, targets):
    loc_t = loc_data.transpose(0, 2, 1)
    conf_t = conf_data.transpose(0, 2, 1)
    pr_t = priors.T
    tg_t = targets.transpose(0, 2, 1)

    comb = pl.pallas_call(
        _stage_a1,
        grid=(B,),
        compiler_params=pltpu.CompilerParams(
            dimension_semantics=("parallel",)),
        in_specs=[
            pl.BlockSpec((4, P), lambda b: (0, 0)),
            pl.BlockSpec((1, NT, 5), lambda b: (b, 0, 0)),
            pl.BlockSpec((1, 5, NT), lambda b: (b, 0, 0)),
        ],
        out_specs=pl.BlockSpec((1, 6, P), lambda b: (b, 0, 0)),
        out_shape=jax.ShapeDtypeStruct((B, 6, P), jnp.float32),
    )(pr_t, targets, tg_t)

    ll, lc, npos = pl.pallas_call(
        _stage_a2,
        grid=(B // RB,),
        compiler_params=pltpu.CompilerParams(
            dimension_semantics=("parallel",)),
        in_specs=[
            pl.BlockSpec((RB, 4, P), lambda b: (b, 0, 0)),
            pl.BlockSpec((RB, C, P), lambda b: (b, 0, 0)),
            pl.BlockSpec((RB, 6, P), lambda b: (b, 0, 0)),
            pl.BlockSpec((4, P), lambda b: (0, 0)),
        ],
        out_specs=[
            pl.BlockSpec((1, 1, 128), lambda b: (b, 0, 0)),
            pl.BlockSpec((1, 1, 128), lambda b: (b, 0, 0)),
            pl.BlockSpec((1, 1, 128), lambda b: (b, 0, 0)),
        ],
        out_shape=[
            jax.ShapeDtypeStruct((B // RB, 1, 128), jnp.float32),
            jax.ShapeDtypeStruct((B // RB, 1, 128), jnp.float32),
            jax.ShapeDtypeStruct((B // RB, 1, 128), jnp.float32),
        ],
    )(loc_t, conf_t, comb, pr_t)

    n = jnp.sum(npos[:, 0, 0]) + 1.0
    return (jnp.sum(ll[:, 0, 0]) / n, jnp.sum(lc[:, 0, 0]) / n)


# speedup vs baseline: 1.5152x; 1.5152x over previous
import functools

import jax
import jax.numpy as jnp
from jax import lax
from jax.experimental import pallas as pl
from jax.experimental.pallas import tpu as pltpu

B = 64
P = 8732
C = 21
NT = 16
RB = 8
_THRESHOLD = 0.5
_NEGPOS_RATIO = 3
_V0 = 0.1
_V1 = 0.2


def _smooth_l1(x):
    ax = jnp.abs(x)
    return jnp.where(ax < 1.0, 0.5 * x * x, ax - 0.5)


def _stage_a1(pr_ref, tg_ref, tgt_ref, comb_ref):
    pr = pr_ref[...]
    t = tg_ref[0]
    t5 = tgt_ref[0]

    px1 = pr[0:1] - pr[2:3] * 0.5
    py1 = pr[1:2] - pr[3:4] * 0.5
    px2 = pr[0:1] + pr[2:3] * 0.5
    py2 = pr[1:2] + pr[3:4] * 0.5

    tx1 = t[:, 0:1]
    ty1 = t[:, 1:2]
    tx2 = t[:, 2:3]
    ty2 = t[:, 3:4]

    iw = jnp.maximum(jnp.minimum(tx2, px2) - jnp.maximum(tx1, px1), 0.0)
    ih = jnp.maximum(jnp.minimum(ty2, py2) - jnp.maximum(ty1, py1), 0.0)
    inter = iw * ih
    area_a = (tx2 - tx1) * (ty2 - ty1)
    area_b = (px2 - px1) * (py2 - py1)
    ov = inter / (area_a + area_b - inter)

    ti = lax.broadcasted_iota(jnp.int32, (NT, P), 0)
    li = lax.broadcasted_iota(jnp.int32, (NT, P), 1)

    bto = jnp.max(ov, axis=0, keepdims=True)
    bti = jnp.min(jnp.where(ov == bto, ti, NT), axis=0, keepdims=True)

    bpo = jnp.max(ov, axis=1, keepdims=True)
    bpi = jnp.min(jnp.where(ov == bpo, li, P), axis=1, keepdims=True)

    eq = li == bpi
    forced_t = jnp.max(jnp.where(eq, ti, -1), axis=0, keepdims=True)
    forced = forced_t >= 0
    bti2 = jnp.where(forced, forced_t, bti)
    bto2 = jnp.where(forced, 2.0, bto)

    oh = (ti == bti2).astype(jnp.float32)
    matched = jnp.dot(t5, oh, preferred_element_type=jnp.float32)

    comb_ref[0, 0:5, :] = matched
    comb_ref[0, 5:6, :] = bto2


def _stage_a2(loc_ref, conf_ref, comb_ref, pr_ref,
              mined_ref, ll_ref, cep_ref, np_ref):
    pr = pr_ref[...]
    cx = pr[0:1]
    cy = pr[1:2]
    w = pr[2:3]
    h = pr[3:4]

    mx1 = comb_ref[:, 0, :]
    my1 = comb_ref[:, 1, :]
    mx2 = comb_ref[:, 2, :]
    my2 = comb_ref[:, 3, :]
    mlab = comb_ref[:, 4, :]
    bto2 = comb_ref[:, 5, :]

    conf = jnp.where(bto2 < _THRESHOLD, 0, mlab.astype(jnp.int32) + 1)
    pos = conf > 0

    gx = ((mx1 + mx2) * 0.5 - cx) / (_V0 * w)
    gy = ((my1 + my2) * 0.5 - cy) / (_V0 * h)
    gw = jnp.log((mx2 - mx1) / w) / _V1
    gh = jnp.log((my2 - my1) / h) / _V1

    sl = (_smooth_l1(loc_ref[:, 0, :] - gx) + _smooth_l1(loc_ref[:, 1, :] - gy)
          + _smooth_l1(loc_ref[:, 2, :] - gw) + _smooth_l1(loc_ref[:, 3, :] - gh))
    loss_l = jnp.sum(jnp.where(pos, sl, 0.0))

    cf = conf_ref[...]
    m = jnp.max(cf, axis=1)
    s = jnp.sum(jnp.exp(cf - m[:, None, :]), axis=1)
    lse = jnp.log(s) + m
    ci = lax.broadcasted_iota(jnp.int32, (RB, C, P), 1)
    gat = jnp.sum(jnp.where(ci == conf[:, None, :], cf, 0.0), axis=1)
    ce = lse - gat

    cep = jnp.sum(jnp.where(pos, ce, 0.0))
    mined = jnp.maximum(jnp.where(pos, 0.0, ce), 0.0)
    npos = jnp.sum(pos.astype(jnp.float32), axis=1, keepdims=True)

    mined_ref[...] = mined
    np_ref[...] = jnp.broadcast_to(npos, (RB, 128))
    ll_ref[...] = jnp.broadcast_to(loss_l, (1, 1, 128))
    cep_ref[...] = jnp.broadcast_to(cep, (1, 1, 128))


def _stage_b(mined_ref, ll_ref, cep_ref, np_ref, o1_ref, o2_ref):
    mined = mined_ref[...]
    keys = lax.bitcast_convert_type(mined, jnp.int32)
    npos = np_ref[...][:, 0:1]
    k = jnp.minimum(_NEGPOS_RATIO * npos.astype(jnp.int32), P - 1)

    def body(i, v):
        bit = jnp.left_shift(jnp.int32(1), 30 - i)
        cand = v | bit
        cnt = jnp.sum((keys >= cand).astype(jnp.int32), axis=1, keepdims=True)
        return jnp.where(cnt >= k, cand, v)

    v = lax.fori_loop(0, 31, body, jnp.zeros((B, 1), jnp.int32))

    gt = keys > v
    cnt_gt = jnp.sum(gt.astype(jnp.int32), axis=1, keepdims=True)
    vf = jnp.max(jnp.where(gt, 0.0, mined), axis=1, keepdims=True)
    topk = (jnp.sum(jnp.where(gt, mined, 0.0), axis=1, keepdims=True)
            + (k - cnt_gt).astype(jnp.float32) * vf)
    topk = jnp.where(k == 0, 0.0, topk)

    loss_c = jnp.sum(cep_ref[...][:, 0:1]) + jnp.sum(topk)
    n = jnp.sum(npos) + 1.0
    loss_l = jnp.sum(ll_ref[...][:, 0:1]) / n
    o1_ref[...] = jnp.broadcast_to(loss_l, (8, 128))
    o2_ref[...] = jnp.broadcast_to(loss_c / n, (8, 128))


@functools.partial(jax.jit, static_argnames=())
def kernel(loc_data, conf_data, priors, targets):
    loc_t = loc_data.transpose(0, 2, 1)
    conf_t = conf_data.transpose(0, 2, 1)
    pr_t = priors.T
    tg_t = targets.transpose(0, 2, 1)

    comb = pl.pallas_call(
        _stage_a1,
        grid=(B,),
        compiler_params=pltpu.CompilerParams(
            dimension_semantics=("parallel",)),
        in_specs=[
            pl.BlockSpec((4, P), lambda b: (0, 0)),
            pl.BlockSpec((1, NT, 5), lambda b: (b, 0, 0)),
            pl.BlockSpec((1, 5, NT), lambda b: (b, 0, 0)),
        ],
        out_specs=pl.BlockSpec((1, 6, P), lambda b: (b, 0, 0)),
        out_shape=jax.ShapeDtypeStruct((B, 6, P), jnp.float32),
    )(pr_t, targets, tg_t)

    mined, ll, cep, npos2 = pl.pallas_call(
        _stage_a2,
        grid=(B // RB,),
        compiler_params=pltpu.CompilerParams(
            dimension_semantics=("parallel",)),
        in_specs=[
            pl.BlockSpec((RB, 4, P), lambda b: (b, 0, 0)),
            pl.BlockSpec((RB, C, P), lambda b: (b, 0, 0)),
            pl.BlockSpec((RB, 6, P), lambda b: (b, 0, 0)),
            pl.BlockSpec((4, P), lambda b: (0, 0)),
        ],
        out_specs=[
            pl.BlockSpec((RB, P), lambda b: (b, 0)),
            pl.BlockSpec((1, 1, 128), lambda b: (b, 0, 0)),
            pl.BlockSpec((1, 1, 128), lambda b: (b, 0, 0)),
            pl.BlockSpec((RB, 128), lambda b: (b, 0)),
        ],
        out_shape=[
            jax.ShapeDtypeStruct((B, P), jnp.float32),
            jax.ShapeDtypeStruct((B // RB, 1, 128), jnp.float32),
            jax.ShapeDtypeStruct((B // RB, 1, 128), jnp.float32),
            jax.ShapeDtypeStruct((B, 128), jnp.float32),
        ],
    )(loc_t, conf_t, comb, pr_t)

    o1, o2 = pl.pallas_call(
        _stage_b,
        in_specs=[
            pl.BlockSpec((B, P), lambda: (0, 0)),
            pl.BlockSpec((B // RB, 128), lambda: (0, 0)),
            pl.BlockSpec((B // RB, 128), lambda: (0, 0)),
            pl.BlockSpec((B, 128), lambda: (0, 0)),
        ],
        out_specs=[
            pl.BlockSpec((8, 128), lambda: (0, 0)),
            pl.BlockSpec((8, 128), lambda: (0, 0)),
        ],
        out_shape=[
            jax.ShapeDtypeStruct((8, 128), jnp.float32),
            jax.ShapeDtypeStruct((8, 128), jnp.float32),
        ],
    )(mined, ll.reshape(B // RB, 128), cep.reshape(B // RB, 128),
      npos2)

    return (o1[0, 0], o2[0, 0])
